# baseline (speedup 1.0000x reference)
import numpy as np
import jax
import jax.numpy as jnp
from jax.experimental import pallas as pl
from jax.experimental.pallas import tpu as pltpu

_B, _L, _H, _D = 2, 2048, 16, 64
_U = 40
_NH = _B * _H
_HP = _H // 2
_QBLK = 256
_NBLK = _L // _QBLK
_SCALE = 1.0 / np.sqrt(_D)
_NEG = np.float32(-3.0e38)

_IDX = np.asarray(
    jax.random.randint(jax.random.key(42), (_L, _U), 0, _L, dtype=jnp.int32)
)
_CNT = np.zeros((_L, _L), dtype=np.float32)
np.add.at(_CNT, (np.repeat(np.arange(_L), _U), _IDX.ravel()), 1.0)
_CNT16 = np.array(jnp.asarray(_CNT, dtype=jnp.bfloat16))
_BIAS_T16 = np.array(
    jnp.asarray(np.where(_CNT.T > 0.0, 0.0, _NEG), dtype=jnp.bfloat16)
)


def _k1_body(bias_ref, cnt_ref, q_ref, k_ref, m_ref):
    q2 = q_ref[:, :]
    k2 = k_ref[:, :]
    q216 = q2.astype(jnp.bfloat16)
    k216 = k2.astype(jnp.bfloat16)
    ks2 = jax.lax.dot_general(
        cnt_ref[:, :], k216, (((1,), (0,)), ((), ())),
        preferred_element_type=jnp.float32,
    )
    for hh in range(2):
        col = slice(hh * _D, (hh + 1) * _D)
        qh16 = q216[:, col]
        kh16 = k216[:, col]
        sm = jnp.sum(q2[:, col] * ks2[:, col], axis=1)
        parts = []
        for j in range(_NBLK):
            qb = qh16[j * _QBLK : (j + 1) * _QBLK, :]
            st = jax.lax.dot_general(
                kh16, qb, (((1,), (1,)), ((), ())),
                preferred_element_type=jnp.float32,
            )
            bb = bias_ref[:, pl.ds(j * _QBLK, _QBLK)].astype(jnp.float32)
            parts.append(jnp.max(st + bb, axis=0))
        mx = jnp.concatenate([p.reshape(1, _QBLK) for p in parts], axis=1)
        m_ref[hh, 0, :] = (mx - sm.reshape(1, _L) * (1.0 / _L))[0]


def _k2_body(m_ref, top_ref):
    m = m_ref[:, 0, :]
    col = jax.lax.broadcasted_iota(jnp.int32, (_NH, _L), 1)
    picks = []
    for _ in range(_U):
        mx = jnp.max(m, axis=1, keepdims=True)
        cand = jnp.where(m == mx, col, jnp.int32(_L))
        idx = jnp.min(cand, axis=1, keepdims=True)
        picks.append(idx)
        m = jnp.where(col == idx, _NEG, m)
    top_ref[:, :] = jnp.concatenate(picks, axis=1)


def _k3_body(top_ref, q_ref, k_ref, v_ref, o_ref):
    for hh in range(2):
        col = slice(hh * _D, (hh + 1) * _D)
        k = k_ref[:, col]
        v = v_ref[:, col]
        rows = [
            q_ref[pl.ds(top_ref[0, 0, hh, u], 1), col] for u in range(_U)
        ]
        qr = jnp.concatenate(rows, axis=0)
        s = jax.lax.dot_general(
            qr.astype(jnp.bfloat16),
            k.astype(jnp.bfloat16),
            (((1,), (1,)), ((), ())),
            preferred_element_type=jnp.float32,
        ) * _SCALE
        s = s - jnp.max(s, axis=1, keepdims=True)
        e = jnp.exp(s)
        a = e / jnp.sum(e, axis=1, keepdims=True)
        upd = jax.lax.dot_general(
            a.astype(jnp.bfloat16),
            v.astype(jnp.bfloat16),
            (((1,), (0,)), ((), ())),
            preferred_element_type=jnp.float32,
        )
        vm = jnp.mean(v, axis=0)
        o_ref[pl.ds(hh * _L, _L), :] = jnp.broadcast_to(vm[None, :], (_L, _D))
        for u in range(_U):
            o_ref[pl.ds(hh * _L + top_ref[0, 0, hh, u], 1), :] = upd[u : u + 1, :]


def _pair_spec():
    return pl.BlockSpec((_L, 2 * _D), lambda b, hp: (b, hp))


def kernel(queries, keys, values, attn_mask):
    del attn_mask
    qf = queries.reshape(_B * _L, _H * _D)
    kf = keys.reshape(_B * _L, _H * _D)
    vf = values.reshape(_B * _L, _H * _D)

    m32 = pl.pallas_call(
        _k1_body,
        grid=(_B, _HP),
        in_specs=[
            pl.BlockSpec((_L, _L), lambda b, hp: (0, 0)),
            pl.BlockSpec((_L, _L), lambda b, hp: (0, 0)),
            _pair_spec(),
            _pair_spec(),
        ],
        out_specs=pl.BlockSpec((2, 1, _L), lambda b, hp: (b * _HP + hp, 0, 0)),
        out_shape=jax.ShapeDtypeStruct((_NH, 1, _L), jnp.float32),
        compiler_params=pltpu.CompilerParams(
            dimension_semantics=("arbitrary", "arbitrary"),
        ),
    )(jnp.asarray(_BIAS_T16), jnp.asarray(_CNT16), qf, kf)

    mtop = pl.pallas_call(
        _k2_body,
        in_specs=[pl.BlockSpec((_NH, 1, _L), lambda: (0, 0, 0))],
        out_specs=pl.BlockSpec((_NH, _U), lambda: (0, 0)),
        out_shape=jax.ShapeDtypeStruct((_NH, _U), jnp.int32),
    )(m32)

    ctx = pl.pallas_call(
        _k3_body,
        grid=(_B, _HP),
        in_specs=[
            pl.BlockSpec(
                (1, 1, 2, _U), lambda b, hp: (b, hp, 0, 0),
                memory_space=pltpu.SMEM,
            ),
            _pair_spec(),
            _pair_spec(),
            _pair_spec(),
        ],
        out_specs=pl.BlockSpec((2 * _L, _D), lambda b, hp: (b * _HP + hp, 0)),
        out_shape=jax.ShapeDtypeStruct((_NH * _L, _D), jnp.float32),
        compiler_params=pltpu.CompilerParams(
            dimension_semantics=("arbitrary", "arbitrary"),
        ),
    )(mtop.reshape(_B, _HP, 2, _U), qf, kf, vf)
    return ctx.reshape(_B, _H, _L, _D)


# speedup vs baseline: 1.3256x; 1.3256x over previous
import numpy as np
import jax
import jax.numpy as jnp
from jax.experimental import pallas as pl
from jax.experimental.pallas import tpu as pltpu

_B, _L, _H, _D = 2, 2048, 16, 64
_U = 40
_NH = _B * _H
_QBLK = 256
_NBLK = _L // _QBLK
_SCALE = 1.0 / np.sqrt(_D)
_NEG = np.float32(-3.0e38)

_IDX = np.asarray(
    jax.random.randint(jax.random.key(42), (_L, _U), 0, _L, dtype=jnp.int32)
)
_CNT_T = np.zeros((_L, _L), dtype=np.float32)
np.add.at(_CNT_T, (_IDX.ravel(), np.repeat(np.arange(_L), _U)), 1.0)


def _k1_body(cnt_ref, q_ref, k_ref, m_ref):
    k16 = k_ref[0, 0, :, :].astype(jnp.bfloat16)
    q16 = q_ref[0, 0, :, :].astype(jnp.bfloat16)
    for j in range(_NBLK):
        qb = q16[j * _QBLK : (j + 1) * _QBLK, :]
        st = jax.lax.dot_general(
            k16, qb, (((1,), (1,)), ((), ())),
            preferred_element_type=jnp.float32,
        )
        cb = cnt_ref[:, pl.ds(j * _QBLK, _QBLK)]
        mx = jnp.max(jnp.where(cb > 0.0, st, _NEG), axis=0)
        sm = jnp.sum(st * cb, axis=0)
        m_ref[0, 0, pl.ds(j * _QBLK, _QBLK)] = mx - sm * (1.0 / _L)


def _k2_body(m_ref, top_ref):
    m = m_ref[:, 0, :]
    col = jax.lax.broadcasted_iota(jnp.int32, (_NH, _L), 1)
    picks = []
    for _ in range(_U):
        mx = jnp.max(m, axis=1, keepdims=True)
        cand = jnp.where(m == mx, col, jnp.int32(_L))
        idx = jnp.min(cand, axis=1, keepdims=True)
        picks.append(idx)
        m = jnp.where(col == idx, _NEG, m)
    top_ref[:, :] = jnp.concatenate(picks, axis=1)


def _k3_body(top_ref, q_ref, k_ref, v_ref, o_ref):
    k = k_ref[0, 0, :, :]
    v = v_ref[0, 0, :, :]
    rows = [q_ref[0, 0, pl.ds(top_ref[0, 0, 0, u], 1), :] for u in range(_U)]
    qr = jnp.concatenate(rows, axis=0)
    s = jax.lax.dot_general(
        qr.astype(jnp.bfloat16),
        k.astype(jnp.bfloat16),
        (((1,), (1,)), ((), ())),
        preferred_element_type=jnp.float32,
    ) * _SCALE
    s = s - jnp.max(s, axis=1, keepdims=True)
    e = jnp.exp(s)
    a = e / jnp.sum(e, axis=1, keepdims=True)
    upd = jax.lax.dot_general(
        a.astype(jnp.bfloat16),
        v.astype(jnp.bfloat16),
        (((1,), (0,)), ((), ())),
        preferred_element_type=jnp.float32,
    )
    vm = jnp.mean(v, axis=0)
    o_ref[0, 0, :, :] = jnp.broadcast_to(vm[None, :], (_L, _D))
    for u in range(_U):
        o_ref[0, 0, pl.ds(top_ref[0, 0, 0, u], 1), :] = upd[u : u + 1, :]


def _qkv_spec():
    return pl.BlockSpec((1, 1, _L, _D), lambda b, h: (b, h, 0, 0))


def kernel(queries, keys, values, attn_mask):
    del attn_mask
    q = jnp.transpose(queries, (0, 2, 1, 3))
    kk = jnp.transpose(keys, (0, 2, 1, 3))
    v = jnp.transpose(values, (0, 2, 1, 3))

    m32 = pl.pallas_call(
        _k1_body,
        grid=(_B, _H),
        in_specs=[
            pl.BlockSpec((_L, _L), lambda b, h: (0, 0)),
            _qkv_spec(),
            _qkv_spec(),
        ],
        out_specs=pl.BlockSpec((1, 1, _L), lambda b, h: (b * _H + h, 0, 0)),
        out_shape=jax.ShapeDtypeStruct((_NH, 1, _L), jnp.float32),
        compiler_params=pltpu.CompilerParams(
            dimension_semantics=("arbitrary", "arbitrary"),
        ),
    )(jnp.asarray(_CNT_T), q, kk)

    mtop = pl.pallas_call(
        _k2_body,
        in_specs=[pl.BlockSpec((_NH, 1, _L), lambda: (0, 0, 0))],
        out_specs=pl.BlockSpec((_NH, _U), lambda: (0, 0)),
        out_shape=jax.ShapeDtypeStruct((_NH, _U), jnp.int32),
    )(m32)

    ctx = pl.pallas_call(
        _k3_body,
        grid=(_B, _H),
        in_specs=[
            pl.BlockSpec(
                (1, 1, 1, _U), lambda b, h: (b, h, 0, 0),
                memory_space=pltpu.SMEM,
            ),
            _qkv_spec(),
            _qkv_spec(),
            _qkv_spec(),
        ],
        out_specs=pl.BlockSpec((1, 1, _L, _D), lambda b, h: (b, h, 0, 0)),
        out_shape=jax.ShapeDtypeStruct((_B, _H, _L, _D), jnp.float32),
        compiler_params=pltpu.CompilerParams(
            dimension_semantics=("arbitrary", "arbitrary"),
        ),
    )(mtop.reshape(_B, _H, 1, _U), q, kk, v)
    return ctx
